# Initial kernel scaffold; baseline (speedup 1.0000x reference)
import jax
import jax.numpy as jnp
from jax.experimental import pallas as pl


def kernel(instruction, emb_table, pos_table):
    raise NotImplementedError("write your pallas kernel here")


# speedup vs baseline: 5.5995x; 5.5995x over previous
import functools

import jax
import jax.numpy as jnp
from jax import lax
from jax.experimental import pallas as pl
from jax.experimental.pallas import tpu as pltpu, tpu_sc as plsc

B = 4096
S = 200
D = 64
N_ROWS = B * S
NC = 2
NS = 16
NW = NC * NS
ROWS_PER_W = N_ROWS // NW
G = 128
CHUNK = 512
NG = CHUNK // G
N_CHUNKS = ROWS_PER_W // CHUNK

_mesh = plsc.VectorSubcoreMesh(
    core_axis_name="c", subcore_axis_name="s", num_cores=NC, num_subcores=NS
)


@functools.partial(
    pl.kernel,
    out_type=jax.ShapeDtypeStruct((N_ROWS, D), jnp.float32),
    mesh=_mesh,
    compiler_params=pltpu.CompilerParams(use_tc_tiling_on_sc=False),
    scratch_types=[
        pltpu.VMEM((S, D), jnp.float32),
        pltpu.VMEM((ROWS_PER_W // G, G), jnp.int32),
        pltpu.VMEM((2, CHUNK, D), jnp.float32),
        pltpu.SemaphoreType.DMA,
        pltpu.SemaphoreType.DMA,
    ],
)
def _embed_sc(idx_hbm, emb_hbm, pos_hbm, out_hbm, pos_v, idx_v, rows_v,
              gsem0, gsem1):
    wid = lax.axis_index("s") * NC + lax.axis_index("c")
    base = wid * ROWS_PER_W

    pltpu.sync_copy(idx_hbm.at[wid], idx_v)
    pltpu.sync_copy(pos_hbm.at[pl.ds(0, S)], pos_v)

    gsems = (gsem0, gsem1)

    def start_chunk(c, buf):
        for j in range(NG):
            pltpu.async_copy(
                emb_hbm.at[idx_v.at[c * NG + j]],
                rows_v.at[buf, pl.ds(j * G, G)],
                gsems[buf],
            )

    def drain_chunk(buf):
        for j in range(NG):
            pltpu.make_async_copy(
                emb_hbm.at[idx_v.at[0]],
                rows_v.at[buf, pl.ds(j * G, G)],
                gsems[buf],
            ).wait()

    def finish_chunk(c, buf):
        row0 = base + c * CHUNK

        def add_rows(r, s0):
            s = s0
            for rr in range(2):
                for d in range(0, D, 16):
                    plsc.addupdate(
                        rows_v.at[buf, r * 2 + rr, pl.ds(d, 16)],
                        pos_v[s, pl.ds(d, 16)],
                    )
                s = lax.rem(s + 1, S)
            return s

        s0 = lax.rem(c * CHUNK, S)
        lax.fori_loop(0, CHUNK // 2, add_rows, s0, unroll=2)
        pltpu.sync_copy(rows_v.at[buf], out_hbm.at[pl.ds(row0, CHUNK)])

    start_chunk(0, 0)

    def pipelined(c, _):
        buf = lax.rem(c, 2)
        nxt = lax.rem(c + 1, 2)

        @pl.when(buf == 0)
        def _():
            start_chunk(c + 1, 1)
            drain_chunk(0)
            finish_chunk(c, 0)

        @pl.when(buf == 1)
        def _():
            start_chunk(c + 1, 0)
            drain_chunk(1)
            finish_chunk(c, 1)

        return 0

    lax.fori_loop(0, N_CHUNKS - 1, pipelined, 0)

    last = (N_CHUNKS - 1) % 2
    drain_chunk(last)
    finish_chunk(N_CHUNKS - 1, last)


def kernel(instruction, emb_table, pos_table):
    b, s = instruction.shape
    idx = instruction.reshape(NW, ROWS_PER_W // G, G).astype(jnp.int32)
    out = _embed_sc(idx, emb_table, pos_table)
    return out.reshape(b, s, D)


# speedup vs baseline: 7.4850x; 1.3367x over previous
import functools

import jax
import jax.numpy as jnp
from jax import lax
from jax.experimental import pallas as pl
from jax.experimental.pallas import tpu as pltpu, tpu_sc as plsc

B = 4096
S = 200
D = 64
NC = 2
NS = 16
NW = NC * NS
B_PER_W = B // NW
CB = 2
GROUPS = ((0, 104), (104, 96))
N_CHUNKS = B_PER_W // CB

_mesh = plsc.VectorSubcoreMesh(
    core_axis_name="c", subcore_axis_name="s", num_cores=NC, num_subcores=NS
)


@functools.partial(
    pl.kernel,
    out_type=jax.ShapeDtypeStruct((B, S, D), jnp.float32),
    mesh=_mesh,
    compiler_params=pltpu.CompilerParams(use_tc_tiling_on_sc=False),
    scratch_types=[
        pltpu.VMEM((S, D), jnp.float32),
        pltpu.VMEM((B_PER_W, S), jnp.int32),
        pltpu.VMEM((2, CB, S, D), jnp.float32),
        pltpu.SemaphoreType.DMA,
        pltpu.SemaphoreType.DMA,
    ],
)
def _embed_sc(idx_hbm, emb_hbm, pos_hbm, out_hbm, pos_v, idx_v, rows_v,
              gsem0, gsem1):
    wid = lax.axis_index("s") * NC + lax.axis_index("c")
    b_base = wid * B_PER_W

    pltpu.sync_copy(idx_hbm.at[pl.ds(b_base, B_PER_W)], idx_v)
    pltpu.sync_copy(pos_hbm.at[pl.ds(0, S)], pos_v)

    gsems = (gsem0, gsem1)

    def start_chunk(c, buf):
        b0 = c * CB
        for q in range(CB):
            for off, n in GROUPS:
                pltpu.async_copy(
                    emb_hbm.at[idx_v.at[b0 + q, pl.ds(off, n)]],
                    rows_v.at[buf, q, pl.ds(off, n)],
                    gsems[buf],
                )

    def drain_chunk(buf):
        for q in range(CB):
            for off, n in GROUPS:
                pltpu.make_async_copy(
                    emb_hbm.at[idx_v.at[0, pl.ds(off, n)]],
                    rows_v.at[buf, q, pl.ds(off, n)],
                    gsems[buf],
                ).wait()

    def finish_chunk(c, buf):
        def add_rows(s, _):
            for d in range(0, D, 16):
                v = pos_v[s, pl.ds(d, 16)]
                for q in range(CB):
                    plsc.addupdate(rows_v.at[buf, q, s, pl.ds(d, 16)], v)
            return 0

        lax.fori_loop(0, S, add_rows, 0, unroll=2)
        pltpu.sync_copy(
            rows_v.at[buf], out_hbm.at[pl.ds(b_base + c * CB, CB)]
        )

    start_chunk(0, 0)

    def pipelined(c, _):
        buf = lax.rem(c, 2)

        @pl.when(buf == 0)
        def _():
            start_chunk(c + 1, 1)
            drain_chunk(0)
            finish_chunk(c, 0)

        @pl.when(buf == 1)
        def _():
            start_chunk(c + 1, 0)
            drain_chunk(1)
            finish_chunk(c, 1)

        return 0

    lax.fori_loop(0, N_CHUNKS - 1, pipelined, 0)

    last = (N_CHUNKS - 1) % 2
    drain_chunk(last)
    finish_chunk(N_CHUNKS - 1, last)


def kernel(instruction, emb_table, pos_table):
    return _embed_sc(instruction.astype(jnp.int32), emb_table, pos_table)


# speedup vs baseline: 14.0062x; 1.8712x over previous
import functools

import jax
import jax.numpy as jnp
from jax import lax
from jax.experimental import pallas as pl
from jax.experimental.pallas import tpu as pltpu, tpu_sc as plsc

B = 4096
S = 200
D = 64
DP = 128
NC = 2
NS = 16
NW = NC * NS
B_PER_W = B // NW
CB = 2
GROUPS = ((0, 104), (104, 96))
N_CHUNKS = B_PER_W // CB

_mesh = plsc.VectorSubcoreMesh(
    core_axis_name="c", subcore_axis_name="s", num_cores=NC, num_subcores=NS
)


@functools.partial(
    pl.kernel,
    out_type=jax.ShapeDtypeStruct((B, S, DP), jnp.float32),
    mesh=_mesh,
    compiler_params=pltpu.CompilerParams(use_tc_tiling_on_sc=False),
    scratch_types=[
        pltpu.VMEM((S * D,), jnp.float32),
        pltpu.VMEM((B_PER_W * S,), jnp.int32),
        pltpu.VMEM((2, CB, S, D), jnp.float32),
        pltpu.SemaphoreType.DMA,
        pltpu.SemaphoreType.DMA,
    ],
)
def _embed_sc(idx_hbm, emb_hbm, pos_hbm, out_hbm, pos_v, idx_v, rows_v,
              gsem0, gsem1):
    wid = lax.axis_index("s") * NC + lax.axis_index("c")
    b_base = wid * B_PER_W

    pltpu.sync_copy(idx_hbm.at[pl.ds(b_base * S, B_PER_W * S)], idx_v)
    pltpu.sync_copy(pos_hbm, pos_v)

    gsems = (gsem0, gsem1)

    def start_chunk(c, buf):
        for q in range(CB):
            for off, n in GROUPS:
                pltpu.async_copy(
                    emb_hbm.at[idx_v.at[pl.ds((c * CB + q) * S + off, n)]],
                    rows_v.at[buf, q, pl.ds(off, n)],
                    gsems[buf],
                )

    def drain_chunk(buf):
        for q in range(CB):
            for off, n in GROUPS:
                pltpu.make_async_copy(
                    emb_hbm.at[idx_v.at[pl.ds(0, n)]],
                    rows_v.at[buf, q, pl.ds(off, n)],
                    gsems[buf],
                ).wait()

    def finish_chunk(c, buf):
        def add_rows(s, _):
            for d in range(0, D, 16):
                v = pos_v[pl.ds(s * D + d, 16)]
                for q in range(CB):
                    plsc.addupdate(rows_v.at[buf, q, s, pl.ds(d, 16)], v)
            return 0

        lax.fori_loop(0, S, add_rows, 0, unroll=2)
        pltpu.sync_copy(
            rows_v.at[buf],
            out_hbm.at[pl.ds(b_base + c * CB, CB), slice(None), pl.ds(0, D)],
        )

    start_chunk(0, 0)

    def pipelined(c, _):
        buf = lax.rem(c, 2)

        @pl.when(buf == 0)
        def _():
            start_chunk(c + 1, 1)
            drain_chunk(0)
            finish_chunk(c, 0)

        @pl.when(buf == 1)
        def _():
            start_chunk(c + 1, 0)
            drain_chunk(1)
            finish_chunk(c, 1)

        return 0

    lax.fori_loop(0, N_CHUNKS - 1, pipelined, 0)

    last = (N_CHUNKS - 1) % 2
    drain_chunk(last)
    finish_chunk(N_CHUNKS - 1, last)


def kernel(instruction, emb_table, pos_table):
    idx = instruction.reshape(-1).astype(jnp.int32)
    pos = pos_table[:S].reshape(-1)
    out = _embed_sc(idx, emb_table, pos)
    return out[..., :D]
